# Initial kernel scaffold; baseline (speedup 1.0000x reference)
import functools

import jax
import jax.numpy as jnp
from jax import lax
from jax.experimental import pallas as pl
from jax.experimental.pallas import tpu as pltpu
from jax.experimental.pallas import tpu_sc as plsc

N_GRAPH, N_NODE, N_FEAT = 256, 128, 9
HIDDEN = 768
NUM_ATOMS_P1 = 4609
NUM_IN_DEG = 512
NUM_OUT_DEG = 512
COMB_ROWS = NUM_ATOMS_P1 + NUM_IN_DEG + NUM_OUT_DEG

NW = 32
GPW = N_GRAPH // NW
NODES_PW = GPW * N_NODE
K = N_FEAT + 2
C = 8
ROWS_PER_STEP = C * K
STEPS = NODES_PW // C
STEPS_PER_GRAPH = N_NODE // C
LANES = 16
NCOL = HIDDEN // LANES


def _body(comb_hbm, idx_hbm, token_hbm, out_hbm, idx_v, gbuf, accb, token_v, sem):
    wid = lax.axis_index("s") * 2 + lax.axis_index("c")
    g0 = wid * GPW

    pltpu.sync_copy(idx_hbm.at[pl.ds(wid * NODES_PW * K, NODES_PW * K)], idx_v)

    pltpu.sync_copy(token_hbm, token_v)
    for g in range(GPW):
        pltpu.sync_copy(token_v, out_hbm.at[g0 + g, pl.ds(0, 1)])

    def step(s, carry):
        pltpu.async_copy(
            comb_hbm.at[idx_v.at[pl.ds(s * ROWS_PER_STEP, ROWS_PER_STEP)]],
            gbuf, sem).wait()

        def nodej(j, _):
            base = j * K

            def colc(c, _):
                cs = pl.ds(c * LANES, LANES)
                a = gbuf[base, cs]
                for t in range(1, K):
                    a = a + gbuf[base + t, cs]
                accb[j, cs] = a
                return 0

            lax.fori_loop(0, NCOL, colc, 0)
            return 0

        lax.fori_loop(0, C, nodej, 0)

        g = g0 + s // STEPS_PER_GRAPH
        n0 = (s % STEPS_PER_GRAPH) * C
        pltpu.sync_copy(accb, out_hbm.at[g, pl.ds(1 + n0, C)])
        return carry

    lax.fori_loop(0, STEPS, step, 0)


def _sc_lookup(comb, idx, graph_token):
    mesh = plsc.VectorSubcoreMesh(core_axis_name="c", subcore_axis_name="s")
    fn = functools.partial(
        pl.kernel,
        mesh=mesh,
        out_type=jax.ShapeDtypeStruct((N_GRAPH, N_NODE + 1, HIDDEN), jnp.float32),
        scratch_types=[
            pltpu.VMEM((NODES_PW * K,), jnp.int32),
            pltpu.VMEM((ROWS_PER_STEP, HIDDEN), jnp.float32),
            pltpu.VMEM((C, HIDDEN), jnp.float32),
            pltpu.VMEM((1, HIDDEN), jnp.float32),
            pltpu.SemaphoreType.DMA,
        ],
    )(_body)
    return fn(comb, idx, graph_token)


def kernel(input_nodes, in_degree, out_degree, atom_table, in_deg_table,
           out_deg_table, graph_token):
    comb = jnp.concatenate([atom_table, in_deg_table, out_deg_table], axis=0)
    idx = jnp.concatenate(
        [
            input_nodes.astype(jnp.int32),
            (in_degree.astype(jnp.int32) + NUM_ATOMS_P1)[..., None],
            (out_degree.astype(jnp.int32) + NUM_ATOMS_P1 + NUM_IN_DEG)[..., None],
        ],
        axis=-1,
    ).reshape(-1)
    return _sc_lookup(comb, idx, graph_token)


# speedup vs baseline: 2.3169x; 2.3169x over previous
import functools

import jax
import jax.numpy as jnp
from jax import lax
from jax.experimental import pallas as pl
from jax.experimental.pallas import tpu as pltpu
from jax.experimental.pallas import tpu_sc as plsc

N_GRAPH, N_NODE, N_FEAT = 256, 128, 9
HIDDEN = 768
NUM_ATOMS_P1 = 4609
NUM_IN_DEG = 512
NUM_OUT_DEG = 512
COMB_ROWS = NUM_ATOMS_P1 + NUM_IN_DEG + NUM_OUT_DEG

NW = 32
GPW = N_GRAPH // NW
NODES_PW = GPW * N_NODE
K = N_FEAT + 2
C = 8
ROWS_PER_STEP = C * K
STEPS = NODES_PW // C
STEPS_PER_GRAPH = N_NODE // C
LANES = 16
NCOL = HIDDEN // LANES


def _body(comb_hbm, idx_hbm, token_hbm, out_hbm, idx_v, gbuf, accb, token_v, sem):
    wid = lax.axis_index("s") * 2 + lax.axis_index("c")
    g0 = wid * GPW

    pltpu.sync_copy(idx_hbm.at[pl.ds(wid * NODES_PW * K, NODES_PW * K)], idx_v)

    pltpu.sync_copy(token_hbm, token_v)
    for g in range(GPW):
        pltpu.sync_copy(token_v, out_hbm.at[pl.ds((g0 + g) * (N_NODE + 1) * HIDDEN, HIDDEN)])

    def step(s, carry):
        pltpu.async_copy(
            comb_hbm.at[idx_v.at[pl.ds(s * ROWS_PER_STEP, ROWS_PER_STEP)]],
            gbuf, sem).wait()

        def nodej(j, _):
            base = j * K

            def colc(c, _):
                cs = pl.ds(c * LANES, LANES)
                a = gbuf[base, cs]
                for t in range(1, K):
                    a = a + gbuf[base + t, cs]
                accb[pl.ds(j * HIDDEN + c * LANES, LANES)] = a
                return 0

            lax.fori_loop(0, NCOL, colc, 0)
            return 0

        lax.fori_loop(0, C, nodej, 0)

        g = g0 + s // STEPS_PER_GRAPH
        n0 = (s % STEPS_PER_GRAPH) * C
        pltpu.sync_copy(
            accb,
            out_hbm.at[pl.ds((g * (N_NODE + 1) + 1 + n0) * HIDDEN, C * HIDDEN)])
        return carry

    lax.fori_loop(0, STEPS, step, 0)


def _sc_lookup(comb, idx, graph_token):
    mesh = plsc.VectorSubcoreMesh(core_axis_name="c", subcore_axis_name="s")
    fn = functools.partial(
        pl.kernel,
        mesh=mesh,
        out_type=jax.ShapeDtypeStruct((N_GRAPH * (N_NODE + 1) * HIDDEN,), jnp.float32),
        scratch_types=[
            pltpu.VMEM((NODES_PW * K,), jnp.int32),
            pltpu.VMEM((ROWS_PER_STEP, HIDDEN), jnp.float32),
            pltpu.VMEM((C * HIDDEN,), jnp.float32),
            pltpu.VMEM((HIDDEN,), jnp.float32),
            pltpu.SemaphoreType.DMA,
        ],
    )(_body)
    return fn(comb, idx, graph_token.reshape(HIDDEN))


def kernel(input_nodes, in_degree, out_degree, atom_table, in_deg_table,
           out_deg_table, graph_token):
    comb = jnp.concatenate([atom_table, in_deg_table, out_deg_table], axis=0)
    idx = jnp.concatenate(
        [
            input_nodes.astype(jnp.int32),
            (in_degree.astype(jnp.int32) + NUM_ATOMS_P1)[..., None],
            (out_degree.astype(jnp.int32) + NUM_ATOMS_P1 + NUM_IN_DEG)[..., None],
        ],
        axis=-1,
    ).reshape(-1)
    flat = _sc_lookup(comb, idx, graph_token)
    return flat.reshape(N_GRAPH, N_NODE + 1, HIDDEN)
